# baseline (speedup 1.0000x reference)
import jax
import jax.numpy as jnp
from jax.experimental import pallas as pl

_B = 16384
_E = 97
_D = 199
_C = 20
_BS = 1024


def _body(x_ref, wt_ref, emb_ref, o_ref):
    xb = x_ref[...]
    lin = jax.lax.dot_general(
        xb, wt_ref[...], (((1,), (0,)), ((), ())),
        preferred_element_type=jnp.float32)
    cat = xb[:, 0:1].astype(jnp.int32)
    iota = jax.lax.broadcasted_iota(jnp.int32, (1, _C + 1), 1)
    onehot = jnp.where(iota == _C, 1.0,
                       (cat == iota).astype(jnp.float32))
    emb = jax.lax.dot_general(
        onehot, emb_ref[...], (((1,), (0,)), ((), ())),
        preferred_element_type=jnp.float32)
    o_ref[...] = lin + emb


def kernel(x, emb_table, W, b):
    wt_pad = jnp.concatenate([jnp.zeros((1, _E), jnp.float32), W.T], axis=0)
    emb2 = jnp.concatenate([emb_table, b.reshape(1, _E)], axis=0)
    return pl.pallas_call(
        _body,
        grid=(_B // _BS,),
        in_specs=[
            pl.BlockSpec((_BS, _D + 1), lambda i: (i, 0)),
            pl.BlockSpec((_D + 1, _E), lambda i: (0, 0)),
            pl.BlockSpec((_C + 1, _E), lambda i: (0, 0)),
        ],
        out_specs=pl.BlockSpec((_BS, _E), lambda i: (i, 0)),
        out_shape=jax.ShapeDtypeStruct((_B, _E), jnp.float32),
    )(x, wt_pad, emb2)


# speedup vs baseline: 1.2184x; 1.2184x over previous
import jax
import jax.numpy as jnp
from jax.experimental import pallas as pl

_B = 16384
_E = 97
_D = 199
_C = 20
_BS = 8192


def _body(x_ref, wt_ref, emb_ref, o_ref):
    xb = x_ref[...]
    lin = jax.lax.dot_general(
        xb, wt_ref[...], (((1,), (0,)), ((), ())),
        preferred_element_type=jnp.float32)
    cat = xb[:, 0:1].astype(jnp.int32)
    iota = jax.lax.broadcasted_iota(jnp.int32, (1, _C + 1), 1)
    onehot = jnp.where(iota == _C, 1.0,
                       (cat == iota).astype(jnp.float32))
    emb = jax.lax.dot_general(
        onehot, emb_ref[...], (((1,), (0,)), ((), ())),
        preferred_element_type=jnp.float32)
    o_ref[...] = lin + emb


def kernel(x, emb_table, W, b):
    wt_pad = jnp.concatenate([jnp.zeros((1, _E), jnp.float32), W.T], axis=0)
    emb2 = jnp.concatenate([emb_table, b.reshape(1, _E)], axis=0)
    return pl.pallas_call(
        _body,
        grid=(_B // _BS,),
        in_specs=[
            pl.BlockSpec((_BS, _D + 1), lambda i: (i, 0)),
            pl.BlockSpec((_D + 1, _E), lambda i: (0, 0)),
            pl.BlockSpec((_C + 1, _E), lambda i: (0, 0)),
        ],
        out_specs=pl.BlockSpec((_BS, _E), lambda i: (i, 0)),
        out_shape=jax.ShapeDtypeStruct((_B, _E), jnp.float32),
    )(x, wt_pad, emb2)
